# Initial kernel scaffold; baseline (speedup 1.0000x reference)
import functools

import jax
import jax.numpy as jnp
from jax import lax
from jax.experimental import pallas as pl
from jax.experimental.pallas import tpu as pltpu
from jax.experimental.pallas import tpu_sc as plsc

N = 10000
D = 128
OUT = 64
E = 320000
NC, NS = 2, 16
NW = NC * NS
CHUNK = 128
E_PAD = 327680
CPW = E_PAD // (NW * CHUNK)
N_PAD = 10240
ROWS_PER_SUB = N_PAD // NS
RB = 2048
GRID = N_PAD // RB

_mesh = plsc.VectorSubcoreMesh(core_axis_name="c", subcore_axis_name="s")


@functools.partial(
    pl.kernel,
    mesh=_mesh,
    out_type=jax.ShapeDtypeStruct((NW, N_PAD), jnp.float32),
    scratch_types=[
        pltpu.VMEM((CPW, CHUNK), jnp.int32),
        pltpu.VMEM((N_PAD,), jnp.float32),
    ],
)
def _sc_degree(dst_hbm, out_hbm, idx_v, hist_v):
    cid = lax.axis_index("c")
    sid = lax.axis_index("s")
    wid = cid * NS + sid
    pltpu.sync_copy(dst_hbm.at[wid], idx_v)
    zeros16 = jnp.zeros((16,), jnp.float32)

    @pl.loop(0, N_PAD // 16)
    def _(i):
        hist_v[pl.ds(i * 16, 16)] = zeros16

    ones16 = jnp.ones((16,), jnp.float32)

    @pl.loop(0, CPW)
    def _(j):
        for i in range(CHUNK // 16):
            idx16 = idx_v[j, pl.ds(i * 16, 16)]
            plsc.addupdate_scatter(hist_v, [idx16], ones16)

    pltpu.sync_copy(hist_v, out_hbm.at[wid])


@functools.partial(
    pl.kernel,
    mesh=_mesh,
    out_type=(
        jax.ShapeDtypeStruct((N_PAD, D), jnp.float32),
        jax.ShapeDtypeStruct((N_PAD, D), jnp.float32),
    ),
    scratch_types=[
        pltpu.VMEM((CPW, CHUNK), jnp.int32),
        pltpu.VMEM((CPW, CHUNK), jnp.int32),
        pltpu.VMEM((CHUNK, D), jnp.float32),
        pltpu.VMEM((CHUNK, D), jnp.float32),
        pltpu.VMEM_SHARED((N_PAD, D), jnp.float32),
        pltpu.SemaphoreType.DMA,
    ],
)
def _sc_matvec(z_hbm, src_hbm, dst_hbm, out_a, out_b,
               src_v, dst_v, rows_v, zeros_v, acc, sem):
    cid = lax.axis_index("c")
    sid = lax.axis_index("s")
    wid = cid * NS + sid

    zeros16 = jnp.zeros((16,), jnp.float32)

    @pl.loop(0, CHUNK)
    def _(r):
        for i in range(D // 16):
            zeros_v[r, pl.ds(i * 16, 16)] = zeros16

    row0 = sid * ROWS_PER_SUB

    @pl.loop(0, ROWS_PER_SUB // CHUNK)
    def _(k):
        pltpu.sync_copy(zeros_v, acc.at[pl.ds(row0 + k * CHUNK, CHUNK)])

    pltpu.sync_copy(src_hbm.at[wid], src_v)
    pltpu.sync_copy(dst_hbm.at[wid], dst_v)
    plsc.subcore_barrier()

    @pl.loop(0, CPW)
    def _(j):
        pltpu.async_copy(z_hbm.at[src_v.at[j]], rows_v, sem).wait()
        pltpu.sync_copy(rows_v, acc.at[dst_v.at[j]], add=True)

    plsc.subcore_barrier()

    @pl.when(cid == 0)
    def _():
        pltpu.sync_copy(acc.at[pl.ds(row0, ROWS_PER_SUB)],
                        out_a.at[pl.ds(row0, ROWS_PER_SUB)])

    @pl.when(cid == 1)
    def _():
        pltpu.sync_copy(acc.at[pl.ds(row0, ROWS_PER_SUB)],
                        out_b.at[pl.ds(row0, ROWS_PER_SUB)])


def _mm1_body(x_ref, w_ref, o_ref):
    o_ref[...] = jnp.dot(x_ref[...], w_ref[...],
                         preferred_element_type=jnp.float32)


_mm1 = pl.pallas_call(
    _mm1_body,
    grid=(GRID,),
    in_specs=[pl.BlockSpec((RB, D), lambda i: (i, 0)),
              pl.BlockSpec((D, D), lambda i: (0, 0))],
    out_specs=pl.BlockSpec((RB, D), lambda i: (i, 0)),
    out_shape=jax.ShapeDtypeStruct((N_PAD, D), jnp.float32),
)


def _scale_body(u_ref, degT_ref, z_ref, dinv_ref):
    deg = jnp.sum(degT_ref[...], axis=1, keepdims=True) + 1.0
    dinv = lax.rsqrt(deg)
    dinv_ref[...] = dinv
    z_ref[...] = u_ref[...] * dinv


_scale = pl.pallas_call(
    _scale_body,
    grid=(GRID,),
    in_specs=[pl.BlockSpec((RB, D), lambda i: (i, 0)),
              pl.BlockSpec((RB, NW), lambda i: (i, 0))],
    out_specs=[pl.BlockSpec((RB, D), lambda i: (i, 0)),
               pl.BlockSpec((RB, 1), lambda i: (i, 0))],
    out_shape=[jax.ShapeDtypeStruct((N_PAD, D), jnp.float32),
               jax.ShapeDtypeStruct((N_PAD, 1), jnp.float32)],
)


def _h_body(ta_ref, tb_ref, z1_ref, dinv_ref, b1_ref, z2_ref):
    i = pl.program_id(0)
    dinv = dinv_ref[...]
    pre = dinv * (ta_ref[...] + tb_ref[...] + z1_ref[...]) + b1_ref[...]
    h = jnp.where(pre >= 0, pre, 0.01 * pre)
    rows = i * RB + lax.broadcasted_iota(jnp.int32, (RB, 1), 0)
    z2_ref[...] = jnp.where(rows < N, dinv * h, 0.0)


_hstage = pl.pallas_call(
    _h_body,
    grid=(GRID,),
    in_specs=[pl.BlockSpec((RB, D), lambda i: (i, 0)),
              pl.BlockSpec((RB, D), lambda i: (i, 0)),
              pl.BlockSpec((RB, D), lambda i: (i, 0)),
              pl.BlockSpec((RB, 1), lambda i: (i, 0)),
              pl.BlockSpec((1, D), lambda i: (0, 0))],
    out_specs=pl.BlockSpec((RB, D), lambda i: (i, 0)),
    out_shape=jax.ShapeDtypeStruct((N_PAD, D), jnp.float32),
)


def _out_body(ta_ref, tb_ref, z2_ref, dinv_ref, w_ref, b_ref, o_ref):
    g = dinv_ref[...] * (ta_ref[...] + tb_ref[...] + z2_ref[...])
    o_ref[...] = jnp.dot(g, w_ref[...],
                         preferred_element_type=jnp.float32) + b_ref[...]


_outstage = pl.pallas_call(
    _out_body,
    grid=(GRID,),
    in_specs=[pl.BlockSpec((RB, D), lambda i: (i, 0)),
              pl.BlockSpec((RB, D), lambda i: (i, 0)),
              pl.BlockSpec((RB, D), lambda i: (i, 0)),
              pl.BlockSpec((RB, 1), lambda i: (i, 0)),
              pl.BlockSpec((D, D), lambda i: (0, 0)),
              pl.BlockSpec((1, D), lambda i: (0, 0))],
    out_specs=pl.BlockSpec((RB, D), lambda i: (i, 0)),
    out_shape=jax.ShapeDtypeStruct((N_PAD, D), jnp.float32),
)


def kernel(x, edge_index, W1, b1, W_mu, b_mu, W_lv, b_lv):
    src = edge_index[0].astype(jnp.int32)
    dst = edge_index[1].astype(jnp.int32)
    pad = jnp.full((E_PAD - E,), N, jnp.int32)
    src3 = jnp.concatenate([src, pad]).reshape(NW, CPW, CHUNK)
    dst3 = jnp.concatenate([dst, pad]).reshape(NW, CPW, CHUNK)
    x_pad = jnp.concatenate(
        [x, jnp.zeros((N_PAD - N, D), jnp.float32)], axis=0)

    hist = _sc_degree(dst3)
    u1 = _mm1(x_pad, W1)
    z1, dinv = _scale(u1, hist.T)
    t1a, t1b = _sc_matvec(z1, src3, dst3)
    z2 = _hstage(t1a, t1b, z1, dinv, b1.reshape(1, D))
    t2a, t2b = _sc_matvec(z2, src3, dst3)
    Wcat = jnp.concatenate([W_mu, W_lv], axis=1)
    bcat = jnp.concatenate([b_mu, b_lv]).reshape(1, D)
    outp = _outstage(t2a, t2b, z2, dinv, Wcat, bcat)
    return outp[:N, :OUT], outp[:N, OUT:]


# speedup vs baseline: 12.0790x; 12.0790x over previous
import dataclasses
import functools

import jax
import jax.numpy as jnp
from jax import lax
from jax.experimental import pallas as pl
from jax.experimental.pallas import tpu as pltpu
from jax.experimental.pallas import tpu_sc as plsc

N = 10000
D = 128
DH = 64
OUT = 64
E = 320000
NC, NS = 2, 16
NW = NC * NS
CHUNK = 128
E_PAD = 327680
NCHUNK = E_PAD // CHUNK
CPW_H = NCHUNK // NW
CPS = NCHUNK // NS
N_PAD = 10240
ROWS_PER_SUB = N_PAD // NS
RB = 2048
GRID = N_PAD // RB

_mesh = plsc.VectorSubcoreMesh(core_axis_name="c", subcore_axis_name="s")

_sc_cp = pltpu.CompilerParams()
if "needs_layout_passes" in pltpu.CompilerParams.__dataclass_fields__:
    _sc_cp = dataclasses.replace(_sc_cp, needs_layout_passes=False)
_sc_cp_mv = dataclasses.replace(_sc_cp, use_tc_tiling_on_sc=False)


@functools.partial(
    pl.kernel,
    mesh=_mesh,
    out_type=jax.ShapeDtypeStruct((NW, N_PAD), jnp.float32),
    compiler_params=_sc_cp,
    scratch_types=[
        pltpu.VMEM((CPW_H, CHUNK), jnp.int32),
        pltpu.VMEM((N_PAD,), jnp.float32),
    ],
)
def _sc_degree(dst_hbm, out_hbm, idx_v, hist_v):
    cid = lax.axis_index("c")
    sid = lax.axis_index("s")
    wid = cid * NS + sid
    pltpu.sync_copy(dst_hbm.at[pl.ds(wid * CPW_H, CPW_H)], idx_v)
    zeros16 = jnp.zeros((16,), jnp.float32)

    @pl.loop(0, N_PAD // 16)
    def _(i):
        hist_v[pl.ds(i * 16, 16)] = zeros16

    ones16 = jnp.ones((16,), jnp.float32)

    @pl.loop(0, CPW_H)
    def _(j):
        for i in range(CHUNK // 16):
            idx16 = idx_v[j, pl.ds(i * 16, 16)]
            plsc.addupdate_scatter(hist_v, [idx16], ones16)

    pltpu.sync_copy(hist_v, out_hbm.at[wid])


@functools.partial(
    pl.kernel,
    mesh=_mesh,
    out_type=(
        jax.ShapeDtypeStruct((N_PAD, DH), jnp.float32),
        jax.ShapeDtypeStruct((N_PAD, DH), jnp.float32),
    ),
    compiler_params=_sc_cp_mv,
    scratch_types=[
        pltpu.VMEM((CPS, CHUNK), jnp.int32),
        pltpu.VMEM((CPS, CHUNK), jnp.int32),
        pltpu.VMEM((CHUNK, DH), jnp.float32),
        pltpu.VMEM((CHUNK, DH), jnp.float32),
        pltpu.VMEM_SHARED((N_PAD, DH), jnp.float32),
        pltpu.SemaphoreType.DMA,
    ],
)
def _sc_matvec(zlo_hbm, zhi_hbm, src_hbm, dst_hbm, out_lo, out_hi,
               src_v, dst_v, rows_v, zeros_v, acc, sem):
    cid = lax.axis_index("c")
    sid = lax.axis_index("s")

    zeros16 = jnp.zeros((16,), jnp.float32)

    @pl.loop(0, CHUNK)
    def _(r):
        for i in range(DH // 16):
            zeros_v[r, pl.ds(i * 16, 16)] = zeros16

    row0 = sid * ROWS_PER_SUB

    @pl.loop(0, ROWS_PER_SUB // CHUNK)
    def _(k):
        pltpu.sync_copy(zeros_v, acc.at[pl.ds(row0 + k * CHUNK, CHUNK)])

    pltpu.sync_copy(src_hbm.at[pl.ds(sid * CPS, CPS)], src_v)
    pltpu.sync_copy(dst_hbm.at[pl.ds(sid * CPS, CPS)], dst_v)
    plsc.subcore_barrier()

    def _run(z_hbm, out_hbm):
        @pl.loop(0, CPS)
        def _(j):
            pltpu.async_copy(z_hbm.at[src_v.at[j]], rows_v, sem).wait()
            pltpu.sync_copy(rows_v, acc.at[dst_v.at[j]], add=True)

        plsc.subcore_barrier()
        pltpu.sync_copy(acc.at[pl.ds(row0, ROWS_PER_SUB)],
                        out_hbm.at[pl.ds(row0, ROWS_PER_SUB)])

    @pl.when(cid == 0)
    def _():
        _run(zlo_hbm, out_lo)

    @pl.when(cid == 1)
    def _():
        _run(zhi_hbm, out_hi)


def _mm1_body(x_ref, w_ref, o_ref):
    o_ref[...] = jnp.dot(x_ref[...], w_ref[...],
                         preferred_element_type=jnp.float32)


_mm1 = pl.pallas_call(
    _mm1_body,
    grid=(GRID,),
    in_specs=[pl.BlockSpec((RB, D), lambda i: (i, 0)),
              pl.BlockSpec((D, D), lambda i: (0, 0))],
    out_specs=pl.BlockSpec((RB, D), lambda i: (i, 0)),
    out_shape=jax.ShapeDtypeStruct((N_PAD, D), jnp.float32),
)


def _scale_body(u_ref, degT_ref, zlo_ref, zhi_ref, dinv_ref):
    deg = jnp.sum(degT_ref[...], axis=1, keepdims=True) + 1.0
    dinv = lax.rsqrt(deg)
    dinv_ref[...] = dinv
    z = u_ref[...] * dinv
    zlo_ref[...] = z[:, :DH]
    zhi_ref[...] = z[:, DH:]


_scale = pl.pallas_call(
    _scale_body,
    grid=(GRID,),
    in_specs=[pl.BlockSpec((RB, D), lambda i: (i, 0)),
              pl.BlockSpec((RB, NW), lambda i: (i, 0))],
    out_specs=[pl.BlockSpec((RB, DH), lambda i: (i, 0)),
               pl.BlockSpec((RB, DH), lambda i: (i, 0)),
               pl.BlockSpec((RB, 1), lambda i: (i, 0))],
    out_shape=[jax.ShapeDtypeStruct((N_PAD, DH), jnp.float32),
               jax.ShapeDtypeStruct((N_PAD, DH), jnp.float32),
               jax.ShapeDtypeStruct((N_PAD, 1), jnp.float32)],
)


def _h_body(tlo_ref, thi_ref, zlo_ref, zhi_ref, dinv_ref, b1_ref,
            olo_ref, ohi_ref):
    i = pl.program_id(0)
    dinv = dinv_ref[...]
    rows = i * RB + lax.broadcasted_iota(jnp.int32, (RB, 1), 0)
    valid = rows < N
    b = b1_ref[...]
    pre_lo = dinv * (tlo_ref[...] + zlo_ref[...]) + b[:, :DH]
    pre_hi = dinv * (thi_ref[...] + zhi_ref[...]) + b[:, DH:]
    h_lo = jnp.where(pre_lo >= 0, pre_lo, 0.01 * pre_lo)
    h_hi = jnp.where(pre_hi >= 0, pre_hi, 0.01 * pre_hi)
    olo_ref[...] = jnp.where(valid, dinv * h_lo, 0.0)
    ohi_ref[...] = jnp.where(valid, dinv * h_hi, 0.0)


_hstage = pl.pallas_call(
    _h_body,
    grid=(GRID,),
    in_specs=[pl.BlockSpec((RB, DH), lambda i: (i, 0)),
              pl.BlockSpec((RB, DH), lambda i: (i, 0)),
              pl.BlockSpec((RB, DH), lambda i: (i, 0)),
              pl.BlockSpec((RB, DH), lambda i: (i, 0)),
              pl.BlockSpec((RB, 1), lambda i: (i, 0)),
              pl.BlockSpec((1, D), lambda i: (0, 0))],
    out_specs=[pl.BlockSpec((RB, DH), lambda i: (i, 0)),
               pl.BlockSpec((RB, DH), lambda i: (i, 0))],
    out_shape=[jax.ShapeDtypeStruct((N_PAD, DH), jnp.float32),
               jax.ShapeDtypeStruct((N_PAD, DH), jnp.float32)],
)


def _out_body(tlo_ref, thi_ref, zlo_ref, zhi_ref, dinv_ref, w_ref, b_ref,
              o_ref):
    dinv = dinv_ref[...]
    g = jnp.concatenate(
        [dinv * (tlo_ref[...] + zlo_ref[...]),
         dinv * (thi_ref[...] + zhi_ref[...])], axis=1)
    o_ref[...] = jnp.dot(g, w_ref[...],
                         preferred_element_type=jnp.float32) + b_ref[...]


_outstage = pl.pallas_call(
    _out_body,
    grid=(GRID,),
    in_specs=[pl.BlockSpec((RB, DH), lambda i: (i, 0)),
              pl.BlockSpec((RB, DH), lambda i: (i, 0)),
              pl.BlockSpec((RB, DH), lambda i: (i, 0)),
              pl.BlockSpec((RB, DH), lambda i: (i, 0)),
              pl.BlockSpec((RB, 1), lambda i: (i, 0)),
              pl.BlockSpec((D, D), lambda i: (0, 0)),
              pl.BlockSpec((1, D), lambda i: (0, 0))],
    out_specs=pl.BlockSpec((RB, D), lambda i: (i, 0)),
    out_shape=jax.ShapeDtypeStruct((N_PAD, D), jnp.float32),
)


def kernel(x, edge_index, W1, b1, W_mu, b_mu, W_lv, b_lv):
    src = edge_index[0].astype(jnp.int32)
    dst = edge_index[1].astype(jnp.int32)
    pad = jnp.full((E_PAD - E,), N, jnp.int32)
    src2 = jnp.concatenate([src, pad]).reshape(NCHUNK, CHUNK)
    dst2 = jnp.concatenate([dst, pad]).reshape(NCHUNK, CHUNK)
    x_pad = jnp.concatenate(
        [x, jnp.zeros((N_PAD - N, D), jnp.float32)], axis=0)

    hist = _sc_degree(dst2)
    u1 = _mm1(x_pad, W1)
    z1lo, z1hi, dinv = _scale(u1, hist.T)
    t1lo, t1hi = _sc_matvec(z1lo, z1hi, src2, dst2)
    z2lo, z2hi = _hstage(t1lo, t1hi, z1lo, z1hi, dinv, b1.reshape(1, D))
    t2lo, t2hi = _sc_matvec(z2lo, z2hi, src2, dst2)
    Wcat = jnp.concatenate([W_mu, W_lv], axis=1)
    bcat = jnp.concatenate([b_mu, b_lv]).reshape(1, D)
    outp = _outstage(t2lo, t2hi, z2lo, z2hi, dinv, Wcat, bcat)
    return outp[:N, :OUT], outp[:N, OUT:]


# speedup vs baseline: 14.0651x; 1.1644x over previous
import dataclasses
import functools

import jax
import jax.numpy as jnp
from jax import lax
from jax.experimental import pallas as pl
from jax.experimental.pallas import tpu as pltpu
from jax.experimental.pallas import tpu_sc as plsc

N = 10000
D = 128
DH = 64
OUT = 64
E = 320000
NC, NS = 2, 16
NW = NC * NS
CHUNK = 128
E_PAD = 327680
NCHUNK = E_PAD // CHUNK
CPW_H = NCHUNK // NW
CPS = NCHUNK // NS
NBUF = 5
N_PAD = 10240
ROWS_PER_SUB = N_PAD // NS
RB = 2048
GRID = N_PAD // RB

_mesh = plsc.VectorSubcoreMesh(core_axis_name="c", subcore_axis_name="s")

_sc_cp = pltpu.CompilerParams()
if "needs_layout_passes" in pltpu.CompilerParams.__dataclass_fields__:
    _sc_cp = dataclasses.replace(_sc_cp, needs_layout_passes=False)
_sc_cp_mv = dataclasses.replace(_sc_cp, use_tc_tiling_on_sc=False)


@functools.partial(
    pl.kernel,
    mesh=_mesh,
    out_type=jax.ShapeDtypeStruct((NW, N_PAD), jnp.float32),
    compiler_params=_sc_cp,
    scratch_types=[
        pltpu.VMEM((CPW_H, CHUNK), jnp.int32),
        pltpu.VMEM((N_PAD,), jnp.float32),
    ],
)
def _sc_degree(dst_hbm, out_hbm, idx_v, hist_v):
    cid = lax.axis_index("c")
    sid = lax.axis_index("s")
    wid = cid * NS + sid
    pltpu.sync_copy(dst_hbm.at[pl.ds(wid * CPW_H, CPW_H)], idx_v)
    zeros16 = jnp.zeros((16,), jnp.float32)

    @pl.loop(0, N_PAD // 16)
    def _(i):
        hist_v[pl.ds(i * 16, 16)] = zeros16

    ones16 = jnp.ones((16,), jnp.float32)

    @pl.loop(0, CPW_H)
    def _(j):
        for i in range(CHUNK // 16):
            idx16 = idx_v[j, pl.ds(i * 16, 16)]
            plsc.addupdate_scatter(hist_v, [idx16], ones16)

    pltpu.sync_copy(hist_v, out_hbm.at[wid])


@functools.partial(
    pl.kernel,
    mesh=_mesh,
    out_type=(
        jax.ShapeDtypeStruct((N_PAD, DH), jnp.float32),
        jax.ShapeDtypeStruct((N_PAD, DH), jnp.float32),
    ),
    compiler_params=_sc_cp_mv,
    scratch_types=[
        pltpu.VMEM((CPS, CHUNK), jnp.int32),
        pltpu.VMEM((CPS, CHUNK), jnp.int32),
        pltpu.VMEM((NBUF, CHUNK, DH), jnp.float32),
        pltpu.VMEM_SHARED((N_PAD, DH), jnp.float32),
        pltpu.SemaphoreType.DMA,
        pltpu.SemaphoreType.DMA,
    ],
)
def _sc_matvec(zlo_hbm, zhi_hbm, src_hbm, dst_hbm, out_lo, out_hi,
               src_v, dst_v, rows_v, acc, sem_g, sem_s):
    cid = lax.axis_index("c")
    sid = lax.axis_index("s")

    zeros16 = jnp.zeros((16,), jnp.float32)

    @pl.loop(0, CHUNK)
    def _(r):
        for i in range(DH // 16):
            rows_v[0, r, pl.ds(i * 16, 16)] = zeros16

    row0 = sid * ROWS_PER_SUB

    @pl.loop(0, ROWS_PER_SUB // CHUNK)
    def _(k):
        pltpu.sync_copy(rows_v.at[0], acc.at[pl.ds(row0 + k * CHUNK, CHUNK)])

    pltpu.sync_copy(src_hbm.at[pl.ds(sid * CPS, CPS)], src_v)
    pltpu.sync_copy(dst_hbm.at[pl.ds(sid * CPS, CPS)], dst_v)
    plsc.subcore_barrier()

    def _run(z_hbm, out_hbm):
        @pl.loop(0, CPS // NBUF)
        def _(g):
            base = g * NBUF
            gathers = [
                pltpu.async_copy(z_hbm.at[src_v.at[base + b]],
                                 rows_v.at[b], sem_g)
                for b in range(NBUF)
            ]
            for c in gathers:
                c.wait()
            scatters = [
                pltpu.async_copy(rows_v.at[b], acc.at[dst_v.at[base + b]],
                                 sem_s, add=True)
                for b in range(NBUF)
            ]
            for c in scatters:
                c.wait()

        plsc.subcore_barrier()
        pltpu.sync_copy(acc.at[pl.ds(row0, ROWS_PER_SUB)],
                        out_hbm.at[pl.ds(row0, ROWS_PER_SUB)])

    @pl.when(cid == 0)
    def _():
        _run(zlo_hbm, out_lo)

    @pl.when(cid == 1)
    def _():
        _run(zhi_hbm, out_hi)


def _mm1_body(x_ref, w_ref, o_ref):
    o_ref[...] = jnp.dot(x_ref[...], w_ref[...],
                         preferred_element_type=jnp.float32)


_mm1 = pl.pallas_call(
    _mm1_body,
    grid=(GRID,),
    in_specs=[pl.BlockSpec((RB, D), lambda i: (i, 0)),
              pl.BlockSpec((D, D), lambda i: (0, 0))],
    out_specs=pl.BlockSpec((RB, D), lambda i: (i, 0)),
    out_shape=jax.ShapeDtypeStruct((N_PAD, D), jnp.float32),
)


def _scale_body(u_ref, degT_ref, zlo_ref, zhi_ref, dinv_ref):
    deg = jnp.sum(degT_ref[...], axis=1, keepdims=True) + 1.0
    dinv = lax.rsqrt(deg)
    dinv_ref[...] = dinv
    z = u_ref[...] * dinv
    zlo_ref[...] = z[:, :DH]
    zhi_ref[...] = z[:, DH:]


_scale = pl.pallas_call(
    _scale_body,
    grid=(GRID,),
    in_specs=[pl.BlockSpec((RB, D), lambda i: (i, 0)),
              pl.BlockSpec((RB, NW), lambda i: (i, 0))],
    out_specs=[pl.BlockSpec((RB, DH), lambda i: (i, 0)),
               pl.BlockSpec((RB, DH), lambda i: (i, 0)),
               pl.BlockSpec((RB, 1), lambda i: (i, 0))],
    out_shape=[jax.ShapeDtypeStruct((N_PAD, DH), jnp.float32),
               jax.ShapeDtypeStruct((N_PAD, DH), jnp.float32),
               jax.ShapeDtypeStruct((N_PAD, 1), jnp.float32)],
)


def _h_body(tlo_ref, thi_ref, zlo_ref, zhi_ref, dinv_ref, b1_ref,
            olo_ref, ohi_ref):
    i = pl.program_id(0)
    dinv = dinv_ref[...]
    rows = i * RB + lax.broadcasted_iota(jnp.int32, (RB, 1), 0)
    valid = rows < N
    b = b1_ref[...]
    pre_lo = dinv * (tlo_ref[...] + zlo_ref[...]) + b[:, :DH]
    pre_hi = dinv * (thi_ref[...] + zhi_ref[...]) + b[:, DH:]
    h_lo = jnp.where(pre_lo >= 0, pre_lo, 0.01 * pre_lo)
    h_hi = jnp.where(pre_hi >= 0, pre_hi, 0.01 * pre_hi)
    olo_ref[...] = jnp.where(valid, dinv * h_lo, 0.0)
    ohi_ref[...] = jnp.where(valid, dinv * h_hi, 0.0)


_hstage = pl.pallas_call(
    _h_body,
    grid=(GRID,),
    in_specs=[pl.BlockSpec((RB, DH), lambda i: (i, 0)),
              pl.BlockSpec((RB, DH), lambda i: (i, 0)),
              pl.BlockSpec((RB, DH), lambda i: (i, 0)),
              pl.BlockSpec((RB, DH), lambda i: (i, 0)),
              pl.BlockSpec((RB, 1), lambda i: (i, 0)),
              pl.BlockSpec((1, D), lambda i: (0, 0))],
    out_specs=[pl.BlockSpec((RB, DH), lambda i: (i, 0)),
               pl.BlockSpec((RB, DH), lambda i: (i, 0))],
    out_shape=[jax.ShapeDtypeStruct((N_PAD, DH), jnp.float32),
               jax.ShapeDtypeStruct((N_PAD, DH), jnp.float32)],
)


def _out_body(tlo_ref, thi_ref, zlo_ref, zhi_ref, dinv_ref, w_ref, b_ref,
              o_ref):
    dinv = dinv_ref[...]
    g = jnp.concatenate(
        [dinv * (tlo_ref[...] + zlo_ref[...]),
         dinv * (thi_ref[...] + zhi_ref[...])], axis=1)
    o_ref[...] = jnp.dot(g, w_ref[...],
                         preferred_element_type=jnp.float32) + b_ref[...]


_outstage = pl.pallas_call(
    _out_body,
    grid=(GRID,),
    in_specs=[pl.BlockSpec((RB, DH), lambda i: (i, 0)),
              pl.BlockSpec((RB, DH), lambda i: (i, 0)),
              pl.BlockSpec((RB, DH), lambda i: (i, 0)),
              pl.BlockSpec((RB, DH), lambda i: (i, 0)),
              pl.BlockSpec((RB, 1), lambda i: (i, 0)),
              pl.BlockSpec((D, D), lambda i: (0, 0)),
              pl.BlockSpec((1, D), lambda i: (0, 0))],
    out_specs=pl.BlockSpec((RB, D), lambda i: (i, 0)),
    out_shape=jax.ShapeDtypeStruct((N_PAD, D), jnp.float32),
)


def kernel(x, edge_index, W1, b1, W_mu, b_mu, W_lv, b_lv):
    src = edge_index[0].astype(jnp.int32)
    dst = edge_index[1].astype(jnp.int32)
    pad = jnp.full((E_PAD - E,), N, jnp.int32)
    src2 = jnp.concatenate([src, pad]).reshape(NCHUNK, CHUNK)
    dst2 = jnp.concatenate([dst, pad]).reshape(NCHUNK, CHUNK)
    x_pad = jnp.concatenate(
        [x, jnp.zeros((N_PAD - N, D), jnp.float32)], axis=0)

    hist = _sc_degree(dst2)
    u1 = _mm1(x_pad, W1)
    z1lo, z1hi, dinv = _scale(u1, hist.T)
    t1lo, t1hi = _sc_matvec(z1lo, z1hi, src2, dst2)
    z2lo, z2hi = _hstage(t1lo, t1hi, z1lo, z1hi, dinv, b1.reshape(1, D))
    t2lo, t2hi = _sc_matvec(z2lo, z2hi, src2, dst2)
    Wcat = jnp.concatenate([W_mu, W_lv], axis=1)
    bcat = jnp.concatenate([b_mu, b_lv]).reshape(1, D)
    outp = _outstage(t2lo, t2hi, z2lo, z2hi, dinv, Wcat, bcat)
    return outp[:N, :OUT], outp[:N, OUT:]


# speedup vs baseline: 15.4369x; 1.0975x over previous
import dataclasses
import functools

import jax
import jax.numpy as jnp
from jax import lax
from jax.experimental import pallas as pl
from jax.experimental.pallas import tpu as pltpu
from jax.experimental.pallas import tpu_sc as plsc

N = 10000
D = 128
DH = 64
OUT = 64
E = 320000
NC, NS = 2, 16
NW = NC * NS
CHUNK = 128
E_PAD = 327680
NCHUNK = E_PAD // CHUNK
CPW_H = NCHUNK // NW
CPS = NCHUNK // NS
NBUF = 5
N_PAD = 10240
ROWS_PER_SUB = N_PAD // NS
RB = 2048
GRID = N_PAD // RB

_mesh = plsc.VectorSubcoreMesh(core_axis_name="c", subcore_axis_name="s")

_sc_cp = pltpu.CompilerParams()
if "needs_layout_passes" in pltpu.CompilerParams.__dataclass_fields__:
    _sc_cp = dataclasses.replace(_sc_cp, needs_layout_passes=False)
_sc_cp_mv = dataclasses.replace(_sc_cp, use_tc_tiling_on_sc=False)


@functools.partial(
    pl.kernel,
    mesh=_mesh,
    out_type=jax.ShapeDtypeStruct((NW, N_PAD), jnp.float32),
    compiler_params=_sc_cp,
    scratch_types=[
        pltpu.VMEM((CPW_H, CHUNK), jnp.int32),
        pltpu.VMEM((N_PAD,), jnp.float32),
    ],
)
def _sc_degree(dst_hbm, out_hbm, idx_v, hist_v):
    cid = lax.axis_index("c")
    sid = lax.axis_index("s")
    wid = cid * NS + sid
    pltpu.sync_copy(dst_hbm.at[pl.ds(wid * CPW_H, CPW_H)], idx_v)
    zeros16 = jnp.zeros((16,), jnp.float32)

    @pl.loop(0, N_PAD // 16)
    def _(i):
        hist_v[pl.ds(i * 16, 16)] = zeros16

    ones16 = jnp.ones((16,), jnp.float32)

    @pl.loop(0, CPW_H)
    def _(j):
        for i in range(CHUNK // 16):
            idx16 = idx_v[j, pl.ds(i * 16, 16)]
            plsc.addupdate_scatter(hist_v, [idx16], ones16)

    pltpu.sync_copy(hist_v, out_hbm.at[wid])


@functools.partial(
    pl.kernel,
    mesh=_mesh,
    out_type=(
        jax.ShapeDtypeStruct((N_PAD, DH), jnp.float32),
        jax.ShapeDtypeStruct((N_PAD, DH), jnp.float32),
    ),
    compiler_params=_sc_cp_mv,
    scratch_types=[
        pltpu.VMEM((CPS, CHUNK), jnp.int32),
        pltpu.VMEM((CPS, CHUNK), jnp.int32),
        pltpu.VMEM((NBUF, CHUNK, DH), jnp.float32),
        pltpu.VMEM_SHARED((N_PAD, DH), jnp.float32),
        pltpu.SemaphoreType.DMA,
        pltpu.SemaphoreType.DMA,
    ],
)
def _sc_matvec(zlo_hbm, zhi_hbm, src_hbm, dst_hbm, out_lo, out_hi,
               src_v, dst_v, rows_v, acc, sem_g, sem_s):
    cid = lax.axis_index("c")
    sid = lax.axis_index("s")

    zeros16 = jnp.zeros((16,), jnp.float32)

    @pl.loop(0, CHUNK)
    def _(r):
        for i in range(DH // 16):
            rows_v[0, r, pl.ds(i * 16, 16)] = zeros16

    row0 = sid * ROWS_PER_SUB

    @pl.loop(0, ROWS_PER_SUB // CHUNK)
    def _(k):
        pltpu.sync_copy(rows_v.at[0], acc.at[pl.ds(row0 + k * CHUNK, CHUNK)])

    pltpu.sync_copy(src_hbm.at[pl.ds(sid * CPS, CPS)], src_v)
    pltpu.sync_copy(dst_hbm.at[pl.ds(sid * CPS, CPS)], dst_v)
    plsc.subcore_barrier()

    def _run(z_hbm, out_hbm):
        def _wait_one(sem):
            pltpu.make_async_copy(z_hbm.at[pl.ds(0, CHUNK)],
                                  rows_v.at[0], sem).wait()

        for b in range(NBUF):
            pltpu.async_copy(z_hbm.at[src_v.at[b]], rows_v.at[b], sem_g)

        @pl.loop(0, CPS // NBUF - 1)
        def _(g):
            base = g * NBUF
            for b in range(NBUF):
                _wait_one(sem_g)
                pltpu.async_copy(rows_v.at[b], acc.at[dst_v.at[base + b]],
                                 sem_s, add=True)
            for b in range(NBUF):
                _wait_one(sem_s)
                pltpu.async_copy(z_hbm.at[src_v.at[base + NBUF + b]],
                                 rows_v.at[b], sem_g)

        base = CPS - NBUF
        for b in range(NBUF):
            _wait_one(sem_g)
            pltpu.async_copy(rows_v.at[b], acc.at[dst_v.at[base + b]],
                             sem_s, add=True)
        for b in range(NBUF):
            _wait_one(sem_s)

        plsc.subcore_barrier()
        pltpu.sync_copy(acc.at[pl.ds(row0, ROWS_PER_SUB)],
                        out_hbm.at[pl.ds(row0, ROWS_PER_SUB)])

    @pl.when(cid == 0)
    def _():
        _run(zlo_hbm, out_lo)

    @pl.when(cid == 1)
    def _():
        _run(zhi_hbm, out_hi)


def _mm1_body(x_ref, w_ref, o_ref):
    o_ref[...] = jnp.dot(x_ref[...], w_ref[...],
                         preferred_element_type=jnp.float32)


_mm1 = pl.pallas_call(
    _mm1_body,
    grid=(GRID,),
    in_specs=[pl.BlockSpec((RB, D), lambda i: (i, 0)),
              pl.BlockSpec((D, D), lambda i: (0, 0))],
    out_specs=pl.BlockSpec((RB, D), lambda i: (i, 0)),
    out_shape=jax.ShapeDtypeStruct((N_PAD, D), jnp.float32),
)


def _scale_body(u_ref, degT_ref, zlo_ref, zhi_ref, dinv_ref):
    deg = jnp.sum(degT_ref[...], axis=1, keepdims=True) + 1.0
    dinv = lax.rsqrt(deg)
    dinv_ref[...] = dinv
    z = u_ref[...] * dinv
    zlo_ref[...] = z[:, :DH]
    zhi_ref[...] = z[:, DH:]


_scale = pl.pallas_call(
    _scale_body,
    grid=(GRID,),
    in_specs=[pl.BlockSpec((RB, D), lambda i: (i, 0)),
              pl.BlockSpec((RB, NW), lambda i: (i, 0))],
    out_specs=[pl.BlockSpec((RB, DH), lambda i: (i, 0)),
               pl.BlockSpec((RB, DH), lambda i: (i, 0)),
               pl.BlockSpec((RB, 1), lambda i: (i, 0))],
    out_shape=[jax.ShapeDtypeStruct((N_PAD, DH), jnp.float32),
               jax.ShapeDtypeStruct((N_PAD, DH), jnp.float32),
               jax.ShapeDtypeStruct((N_PAD, 1), jnp.float32)],
)


def _h_body(tlo_ref, thi_ref, zlo_ref, zhi_ref, dinv_ref, b1_ref,
            olo_ref, ohi_ref):
    i = pl.program_id(0)
    dinv = dinv_ref[...]
    rows = i * RB + lax.broadcasted_iota(jnp.int32, (RB, 1), 0)
    valid = rows < N
    b = b1_ref[...]
    pre_lo = dinv * (tlo_ref[...] + zlo_ref[...]) + b[:, :DH]
    pre_hi = dinv * (thi_ref[...] + zhi_ref[...]) + b[:, DH:]
    h_lo = jnp.where(pre_lo >= 0, pre_lo, 0.01 * pre_lo)
    h_hi = jnp.where(pre_hi >= 0, pre_hi, 0.01 * pre_hi)
    olo_ref[...] = jnp.where(valid, dinv * h_lo, 0.0)
    ohi_ref[...] = jnp.where(valid, dinv * h_hi, 0.0)


_hstage = pl.pallas_call(
    _h_body,
    grid=(GRID,),
    in_specs=[pl.BlockSpec((RB, DH), lambda i: (i, 0)),
              pl.BlockSpec((RB, DH), lambda i: (i, 0)),
              pl.BlockSpec((RB, DH), lambda i: (i, 0)),
              pl.BlockSpec((RB, DH), lambda i: (i, 0)),
              pl.BlockSpec((RB, 1), lambda i: (i, 0)),
              pl.BlockSpec((1, D), lambda i: (0, 0))],
    out_specs=[pl.BlockSpec((RB, DH), lambda i: (i, 0)),
               pl.BlockSpec((RB, DH), lambda i: (i, 0))],
    out_shape=[jax.ShapeDtypeStruct((N_PAD, DH), jnp.float32),
               jax.ShapeDtypeStruct((N_PAD, DH), jnp.float32)],
)


def _out_body(tlo_ref, thi_ref, zlo_ref, zhi_ref, dinv_ref, w_ref, b_ref,
              o_ref):
    dinv = dinv_ref[...]
    g = jnp.concatenate(
        [dinv * (tlo_ref[...] + zlo_ref[...]),
         dinv * (thi_ref[...] + zhi_ref[...])], axis=1)
    o_ref[...] = jnp.dot(g, w_ref[...],
                         preferred_element_type=jnp.float32) + b_ref[...]


_outstage = pl.pallas_call(
    _out_body,
    grid=(GRID,),
    in_specs=[pl.BlockSpec((RB, DH), lambda i: (i, 0)),
              pl.BlockSpec((RB, DH), lambda i: (i, 0)),
              pl.BlockSpec((RB, DH), lambda i: (i, 0)),
              pl.BlockSpec((RB, DH), lambda i: (i, 0)),
              pl.BlockSpec((RB, 1), lambda i: (i, 0)),
              pl.BlockSpec((D, D), lambda i: (0, 0)),
              pl.BlockSpec((1, D), lambda i: (0, 0))],
    out_specs=pl.BlockSpec((RB, D), lambda i: (i, 0)),
    out_shape=jax.ShapeDtypeStruct((N_PAD, D), jnp.float32),
)


def kernel(x, edge_index, W1, b1, W_mu, b_mu, W_lv, b_lv):
    src = edge_index[0].astype(jnp.int32)
    dst = edge_index[1].astype(jnp.int32)
    pad = jnp.full((E_PAD - E,), N, jnp.int32)
    src2 = jnp.concatenate([src, pad]).reshape(NCHUNK, CHUNK)
    dst2 = jnp.concatenate([dst, pad]).reshape(NCHUNK, CHUNK)
    x_pad = jnp.concatenate(
        [x, jnp.zeros((N_PAD - N, D), jnp.float32)], axis=0)

    hist = _sc_degree(dst2)
    u1 = _mm1(x_pad, W1)
    z1lo, z1hi, dinv = _scale(u1, hist.T)
    t1lo, t1hi = _sc_matvec(z1lo, z1hi, src2, dst2)
    z2lo, z2hi = _hstage(t1lo, t1hi, z1lo, z1hi, dinv, b1.reshape(1, D))
    t2lo, t2hi = _sc_matvec(z2lo, z2hi, src2, dst2)
    Wcat = jnp.concatenate([W_mu, W_lv], axis=1)
    bcat = jnp.concatenate([b_mu, b_lv]).reshape(1, D)
    outp = _outstage(t2lo, t2hi, z2lo, z2hi, dinv, Wcat, bcat)
    return outp[:N, :OUT], outp[:N, OUT:]
